# baseline (speedup 1.0000x reference)
import functools

import jax
import jax.numpy as jnp
from jax import lax
from jax.experimental import pallas as pl
from jax.experimental.pallas import tpu as pltpu
from jax.experimental.pallas import tpu_sc as plsc

N = 10000
E = 320000
D_IN = 128
H1 = 64
H2 = 16

NC = 2
NS = 16
NW = NC * NS
E_PAD = 327680
EROWS = E_PAD // 128
EROWS_W = EROWS // NW
CHUNKS = EROWS_W // 8
N_ACC = 10112
ROWS_T = N_ACC // NS
SPLIT_C0 = 80
SPLIT_C1 = 80


def _sc_deg_body(dst_hbm, ones_hbm, zeros_hbm, out_hbm, idx_v, ones_v, acc, sem):
    c = lax.axis_index("c")
    s = lax.axis_index("s")
    wid = s * NC + c
    pltpu.sync_copy(zeros_hbm.at[pl.ds(s * ROWS_T, ROWS_T)],
                    acc.at[pl.ds(s * ROWS_T, ROWS_T)])
    pltpu.sync_copy(ones_hbm, ones_v)
    plsc.subcore_barrier()
    pltpu.sync_copy(dst_hbm.at[pl.ds(wid * EROWS_W, EROWS_W)], idx_v)

    def body(j, carry):
        pltpu.sync_copy(ones_v, acc.at[idx_v.at[j]], add=True)
        return carry

    lax.fori_loop(0, EROWS_W, body, 0)
    plsc.subcore_barrier()
    pltpu.sync_copy(acc.at[pl.ds(s * ROWS_T, ROWS_T)],
                    out_hbm.at[c, pl.ds(s * ROWS_T, ROWS_T)])


def _sc_agg_body(src_hbm, dst_hbm, table_hbm, zeros_hbm, out_hbm,
                 isrc, idst, rows_v, acc,
                 g0, g1, g2, g3, g4, g5, g6, g7,
                 s0, s1, s2, s3, s4, s5, s6, s7):
    c = lax.axis_index("c")
    s = lax.axis_index("s")
    gsem = (g0, g1, g2, g3, g4, g5, g6, g7)
    ssem = (s0, s1, s2, s3, s4, s5, s6, s7)
    pltpu.sync_copy(zeros_hbm.at[pl.ds(s * ROWS_T, ROWS_T)],
                    acc.at[pl.ds(s * ROWS_T, ROWS_T)])
    plsc.subcore_barrier()

    def fire_gather(k, b):
        pltpu.async_copy(table_hbm.at[isrc.at[k]],
                         rows_v.at[pl.ds(b * 128, 128)], gsem[b])

    def gather_done(k, b):
        pltpu.make_async_copy(table_hbm.at[isrc.at[k]],
                              rows_v.at[pl.ds(b * 128, 128)], gsem[b]).wait()

    def fire_scatter(k, b):
        pltpu.async_copy(rows_v.at[pl.ds(b * 128, 128)],
                         acc.at[idst.at[k]], ssem[b], add=True)

    def scatter_done(k, b):
        pltpu.make_async_copy(rows_v.at[pl.ds(b * 128, 128)],
                              acc.at[idst.at[k]], ssem[b]).wait()

    def pipeline(hbm_base, n_rows):
        pltpu.sync_copy(src_hbm.at[pl.ds(hbm_base, n_rows)],
                        isrc.at[pl.ds(0, n_rows)])
        pltpu.sync_copy(dst_hbm.at[pl.ds(hbm_base, n_rows)],
                        idst.at[pl.ds(0, n_rows)])
        for b in range(4):
            fire_gather(b, b)

        def round_(r, carry):
            for b in range(8):
                k = 8 * r + b
                b2 = (b + 4) % 8
                gather_done(k, b)
                fire_scatter(k, b)

                @pl.when(k >= 4)
                def _():
                    scatter_done(k - 4, b2)

                @pl.when(k + 4 < n_rows)
                def _():
                    fire_gather(k + 4, b2)

            return carry

        lax.fori_loop(0, n_rows // 8, round_, 0)
        for k in range(n_rows - 4, n_rows):
            scatter_done(k, k % 8)

    if SPLIT_C0 > 0:
        @pl.when(c == 0)
        def _():
            pipeline(s * 160, SPLIT_C0)
    if SPLIT_C1 > 0:
        @pl.when(c == 1)
        def _():
            pipeline(s * 160 + SPLIT_C0, SPLIT_C1)
    plsc.subcore_barrier()
    pltpu.sync_copy(acc.at[pl.ds(s * ROWS_T, ROWS_T)],
                    out_hbm.at[c, pl.ds(s * ROWS_T, ROWS_T)])


@functools.lru_cache(maxsize=None)
def _sc_kernels():
    mesh = plsc.VectorSubcoreMesh(core_axis_name="c", subcore_axis_name="s")
    params = pltpu.CompilerParams(use_tc_tiling_on_sc=False)
    sc_deg = pl.kernel(
        _sc_deg_body,
        out_type=jax.ShapeDtypeStruct((NC, N_ACC, 16), jnp.float32),
        mesh=mesh,
        compiler_params=params,
        scratch_types=[
            pltpu.VMEM((EROWS_W, 128), jnp.int32),
            pltpu.VMEM((128, 16), jnp.float32),
            pltpu.VMEM_SHARED((N_ACC, 16), jnp.float32),
            pltpu.SemaphoreType.DMA,
        ],
    )
    def make_agg(width):
        return pl.kernel(
            _sc_agg_body,
            out_type=jax.ShapeDtypeStruct((NC, N_ACC, width), jnp.float32),
            mesh=mesh,
            compiler_params=params,
            scratch_types=[
                pltpu.VMEM((80, 128), jnp.int32),
                pltpu.VMEM((80, 128), jnp.int32),
                pltpu.VMEM((1024, width), jnp.float32),
                pltpu.VMEM_SHARED((N_ACC, width), jnp.float32),
            ] + [pltpu.SemaphoreType.DMA] * 16,
        )

    return sc_deg, make_agg(H1), make_agg(2 * H2)


def _dinv_from(degp, n_rows):
    deg = degp[0][:n_rows, 0:1] + degp[1][:n_rows, 0:1] + 1.0
    return lax.rsqrt(deg)


def _tc_mm_body(x_ref, w_ref, h_ref):
    h_ref[...] = jnp.dot(x_ref[...], w_ref[...],
                         preferred_element_type=jnp.float32)


def _tc_scale_body(degp_ref, h_ref, g_ref):
    dinv = _dinv_from(degp_ref[...], N)
    g_ref[...] = h_ref[...] * dinv


def _tc_h1_body(degp_ref, aggp_ref, g_ref, b_ref, wn_ref, h1_ref, t2_ref):
    dinv = _dinv_from(degp_ref[...], N)
    aggp = aggp_ref[...]
    agg = aggp[0][:N] + aggp[1][:N]
    h1 = dinv * (agg + g_ref[...]) + b_ref[...]
    h1_ref[...] = h1
    t2_ref[...] = jnp.dot(h1, wn_ref[...], preferred_element_type=jnp.float32)


def _tc_head_body(h1_ref, aggp_ref, wmr_ref, bm_ref,
                  wvr_ref, bv_ref, mu_ref, lv_ref):
    aggp = aggp_ref[...]
    agg = aggp[0][:N] + aggp[1][:N]
    h1 = h1_ref[...]
    mu_ref[...] = (jnp.dot(h1, wmr_ref[...], preferred_element_type=jnp.float32)
                   + agg[:, :H2] + bm_ref[...])
    lv_ref[...] = (jnp.dot(h1, wvr_ref[...], preferred_element_type=jnp.float32)
                   + agg[:, H2:] + bv_ref[...])


_DEC_B = 1024


def _tc_dec_body(a_ref, b_ref, o_ref):
    v = lax.dot_general(a_ref[...], b_ref[...],
                        (((1,), (1,)), ((), ())),
                        preferred_element_type=jnp.float32)
    o_ref[...] = jax.nn.sigmoid(v)


def _tc_mm(x, w):
    return pl.pallas_call(
        _tc_mm_body,
        out_shape=jax.ShapeDtypeStruct((N, H1), jnp.float32),
    )(x, w)


def _tc_scale(degp, h):
    return pl.pallas_call(
        _tc_scale_body,
        out_shape=jax.ShapeDtypeStruct((N, H1), jnp.float32),
    )(degp, h)


def _tc_h1(degp, aggp, g, b, wn):
    return pl.pallas_call(
        _tc_h1_body,
        out_shape=(jax.ShapeDtypeStruct((N, H1), jnp.float32),
                   jax.ShapeDtypeStruct((N, 2 * H2), jnp.float32)),
    )(degp, aggp, g, b, wn)


def _tc_head(h1, aggp, wmr, bm, wvr, bv):
    return pl.pallas_call(
        _tc_head_body,
        out_shape=(jax.ShapeDtypeStruct((N, H2), jnp.float32),
                   jax.ShapeDtypeStruct((N, H2), jnp.float32)),
    )(h1, aggp, wmr, bm, wvr, bv)


def _tc_dec(z):
    grid = (pl.cdiv(N, _DEC_B), pl.cdiv(N, _DEC_B))
    return pl.pallas_call(
        _tc_dec_body,
        grid=grid,
        in_specs=[
            pl.BlockSpec((_DEC_B, H2), lambda i, j: (i, 0)),
            pl.BlockSpec((_DEC_B, H2), lambda i, j: (j, 0)),
        ],
        out_specs=pl.BlockSpec((_DEC_B, _DEC_B), lambda i, j: (i, j)),
        out_shape=jax.ShapeDtypeStruct((N, N), jnp.float32),
    )(z, z)


def kernel(x, edge_index, W_gc1, b_gc1, Wm_root, Wm_nbr, bm, Wv_root, Wv_nbr, bv):
    src = edge_index[0]
    dst = edge_index[1]
    pad = E_PAD - E
    src_p = jnp.concatenate(
        [src, jnp.zeros((pad,), jnp.int32)]).reshape(EROWS, 128)
    dst_p = jnp.concatenate(
        [dst, jnp.full((pad,), N, jnp.int32)]).reshape(EROWS, 128)
    zeros64 = jnp.zeros((N_ACC, H1), jnp.float32)
    zeros32 = jnp.zeros((N_ACC, 2 * H2), jnp.float32)
    zeros16 = jnp.zeros((N_ACC, 16), jnp.float32)
    ones16 = jnp.zeros((128, 16), jnp.float32).at[:, 0].set(1.0)
    wn = jnp.concatenate([Wm_nbr, Wv_nbr], axis=1)

    sc_deg, sc_agg64, sc_agg32 = _sc_kernels()
    h = _tc_mm(x, W_gc1)
    degp = sc_deg(dst_p, ones16, zeros16)
    g = _tc_scale(degp, h)
    agg1p = sc_agg64(src_p, dst_p, g, zeros64)
    h1, t2 = _tc_h1(degp, agg1p, g, b_gc1.reshape(1, H1), wn)
    agg2p = sc_agg32(src_p, dst_p, t2, zeros32)
    mu, logvar = _tc_head(h1, agg2p, Wm_root, bm.reshape(1, H2),
                          Wv_root, bv.reshape(1, H2))
    adj = _tc_dec(mu)
    return (adj, mu, mu, logvar)


# speedup vs baseline: 1.0094x; 1.0094x over previous
import functools

import jax
import jax.numpy as jnp
from jax import lax
from jax.experimental import pallas as pl
from jax.experimental.pallas import tpu as pltpu
from jax.experimental.pallas import tpu_sc as plsc

N = 10000
E = 320000
D_IN = 128
H1 = 64
H2 = 16

NC = 2
NS = 16
NW = NC * NS
E_PAD = 327680
EROWS = E_PAD // 128
EROWS_W = EROWS // NW
CHUNKS = EROWS_W // 8
N_ACC = 10112
ROWS_T = N_ACC // NS
SPLIT_C0 = 80
SPLIT_C1 = 80


def _sc_deg_body(dst_hbm, ones_hbm, zeros_hbm, out_hbm, idx_v, ones_v, acc, sem):
    c = lax.axis_index("c")
    s = lax.axis_index("s")
    wid = s * NC + c
    pltpu.sync_copy(zeros_hbm.at[pl.ds(s * ROWS_T, ROWS_T)],
                    acc.at[pl.ds(s * ROWS_T, ROWS_T)])
    pltpu.sync_copy(ones_hbm, ones_v)
    plsc.subcore_barrier()
    pltpu.sync_copy(dst_hbm.at[pl.ds(wid * EROWS_W, EROWS_W)], idx_v)

    def body(j, carry):
        pltpu.sync_copy(ones_v, acc.at[idx_v.at[j]], add=True)
        return carry

    lax.fori_loop(0, EROWS_W, body, 0)
    plsc.subcore_barrier()
    pltpu.sync_copy(acc.at[pl.ds(s * ROWS_T, ROWS_T)],
                    out_hbm.at[c, pl.ds(s * ROWS_T, ROWS_T)])


def _sc_agg_body(src_hbm, dst_hbm, table_hbm, zeros_hbm, out_hbm,
                 isrc, idst, rows_v, acc,
                 g0, g1, g2, g3, s0, s1, s2, s3):
    c = lax.axis_index("c")
    s = lax.axis_index("s")
    gsem = (g0, g1, g2, g3)
    ssem = (s0, s1, s2, s3)
    pltpu.sync_copy(zeros_hbm.at[pl.ds(s * ROWS_T, ROWS_T)],
                    acc.at[pl.ds(s * ROWS_T, ROWS_T)])
    plsc.subcore_barrier()

    def fire_gather(k, b):
        pltpu.async_copy(table_hbm.at[isrc.at[k]],
                         rows_v.at[pl.ds(b * 128, 128)], gsem[b])

    def gather_done(k, b):
        pltpu.make_async_copy(table_hbm.at[isrc.at[k]],
                              rows_v.at[pl.ds(b * 128, 128)], gsem[b]).wait()

    def fire_scatter(k, b):
        pltpu.async_copy(rows_v.at[pl.ds(b * 128, 128)],
                         acc.at[idst.at[k]], ssem[b], add=True)

    def scatter_done(k, b):
        pltpu.make_async_copy(rows_v.at[pl.ds(b * 128, 128)],
                              acc.at[idst.at[k]], ssem[b]).wait()

    def pipeline(hbm_base, n_rows):
        pltpu.sync_copy(src_hbm.at[pl.ds(hbm_base, n_rows)],
                        isrc.at[pl.ds(0, n_rows)])
        pltpu.sync_copy(dst_hbm.at[pl.ds(hbm_base, n_rows)],
                        idst.at[pl.ds(0, n_rows)])
        for b in range(2):
            fire_gather(b, b)

        def round_(r, carry):
            for b in range(4):
                k = 4 * r + b
                b2 = (b + 2) % 4
                gather_done(k, b)
                fire_scatter(k, b)

                @pl.when(k >= 2)
                def _():
                    scatter_done(k - 2, b2)

                @pl.when(k + 2 < n_rows)
                def _():
                    fire_gather(k + 2, b2)

            return carry

        lax.fori_loop(0, n_rows // 4, round_, 0)
        for k in range(n_rows - 2, n_rows):
            scatter_done(k, k % 4)

    if SPLIT_C0 > 0:
        @pl.when(c == 0)
        def _():
            pipeline(s * 160, SPLIT_C0)
    if SPLIT_C1 > 0:
        @pl.when(c == 1)
        def _():
            pipeline(s * 160 + SPLIT_C0, SPLIT_C1)
    plsc.subcore_barrier()
    pltpu.sync_copy(acc.at[pl.ds(s * ROWS_T, ROWS_T)],
                    out_hbm.at[c, pl.ds(s * ROWS_T, ROWS_T)])


@functools.lru_cache(maxsize=None)
def _sc_kernels():
    mesh = plsc.VectorSubcoreMesh(core_axis_name="c", subcore_axis_name="s")
    params = pltpu.CompilerParams(use_tc_tiling_on_sc=False)
    sc_deg = pl.kernel(
        _sc_deg_body,
        out_type=jax.ShapeDtypeStruct((NC, N_ACC, 16), jnp.float32),
        mesh=mesh,
        compiler_params=params,
        scratch_types=[
            pltpu.VMEM((EROWS_W, 128), jnp.int32),
            pltpu.VMEM((128, 16), jnp.float32),
            pltpu.VMEM_SHARED((N_ACC, 16), jnp.float32),
            pltpu.SemaphoreType.DMA,
        ],
    )
    def make_agg(width):
        return pl.kernel(
            _sc_agg_body,
            out_type=jax.ShapeDtypeStruct((NC, N_ACC, width), jnp.float32),
            mesh=mesh,
            compiler_params=params,
            scratch_types=[
                pltpu.VMEM((80, 128), jnp.int32),
                pltpu.VMEM((80, 128), jnp.int32),
                pltpu.VMEM((512, width), jnp.float32),
                pltpu.VMEM_SHARED((N_ACC, width), jnp.float32),
            ] + [pltpu.SemaphoreType.DMA] * 8,
        )

    return sc_deg, make_agg(H1), make_agg(2 * H2)


def _dinv_from(degp, n_rows):
    deg = degp[0][:n_rows, 0:1] + degp[1][:n_rows, 0:1] + 1.0
    return lax.rsqrt(deg)


def _tc_mm_body(x_ref, w_ref, h_ref):
    h_ref[...] = jnp.dot(x_ref[...], w_ref[...],
                         preferred_element_type=jnp.float32)


def _tc_scale_body(degp_ref, h_ref, g_ref):
    dinv = _dinv_from(degp_ref[...], N)
    g_ref[...] = h_ref[...] * dinv


def _tc_h1_body(degp_ref, aggp_ref, g_ref, b_ref, wn_ref, h1_ref, t2_ref):
    dinv = _dinv_from(degp_ref[...], N)
    aggp = aggp_ref[...]
    agg = aggp[0][:N] + aggp[1][:N]
    h1 = dinv * (agg + g_ref[...]) + b_ref[...]
    h1_ref[...] = h1
    t2_ref[...] = jnp.dot(h1, wn_ref[...], preferred_element_type=jnp.float32)


def _tc_head_body(h1_ref, aggp_ref, wmr_ref, bm_ref,
                  wvr_ref, bv_ref, mu_ref, lv_ref):
    aggp = aggp_ref[...]
    agg = aggp[0][:N] + aggp[1][:N]
    h1 = h1_ref[...]
    mu_ref[...] = (jnp.dot(h1, wmr_ref[...], preferred_element_type=jnp.float32)
                   + agg[:, :H2] + bm_ref[...])
    lv_ref[...] = (jnp.dot(h1, wvr_ref[...], preferred_element_type=jnp.float32)
                   + agg[:, H2:] + bv_ref[...])


_DEC_B = 1024


def _tc_dec_body(a_ref, b_ref, o_ref):
    v = lax.dot_general(a_ref[...], b_ref[...],
                        (((1,), (1,)), ((), ())),
                        preferred_element_type=jnp.float32)
    o_ref[...] = jax.nn.sigmoid(v)


def _tc_mm(x, w):
    return pl.pallas_call(
        _tc_mm_body,
        out_shape=jax.ShapeDtypeStruct((N, H1), jnp.float32),
    )(x, w)


def _tc_scale(degp, h):
    return pl.pallas_call(
        _tc_scale_body,
        out_shape=jax.ShapeDtypeStruct((N, H1), jnp.float32),
    )(degp, h)


def _tc_h1(degp, aggp, g, b, wn):
    return pl.pallas_call(
        _tc_h1_body,
        out_shape=(jax.ShapeDtypeStruct((N, H1), jnp.float32),
                   jax.ShapeDtypeStruct((N, 2 * H2), jnp.float32)),
    )(degp, aggp, g, b, wn)


def _tc_head(h1, aggp, wmr, bm, wvr, bv):
    return pl.pallas_call(
        _tc_head_body,
        out_shape=(jax.ShapeDtypeStruct((N, H2), jnp.float32),
                   jax.ShapeDtypeStruct((N, H2), jnp.float32)),
    )(h1, aggp, wmr, bm, wvr, bv)


def _tc_dec(z):
    grid = (pl.cdiv(N, _DEC_B), pl.cdiv(N, _DEC_B))
    return pl.pallas_call(
        _tc_dec_body,
        grid=grid,
        in_specs=[
            pl.BlockSpec((_DEC_B, H2), lambda i, j: (i, 0)),
            pl.BlockSpec((_DEC_B, H2), lambda i, j: (j, 0)),
        ],
        out_specs=pl.BlockSpec((_DEC_B, _DEC_B), lambda i, j: (i, j)),
        out_shape=jax.ShapeDtypeStruct((N, N), jnp.float32),
    )(z, z)


def kernel(x, edge_index, W_gc1, b_gc1, Wm_root, Wm_nbr, bm, Wv_root, Wv_nbr, bv):
    src = edge_index[0]
    dst = edge_index[1]
    pad = E_PAD - E
    src_p = jnp.concatenate(
        [src, jnp.zeros((pad,), jnp.int32)]).reshape(EROWS, 128)
    dst_p = jnp.concatenate(
        [dst, jnp.full((pad,), N, jnp.int32)]).reshape(EROWS, 128)
    zeros64 = jnp.zeros((N_ACC, H1), jnp.float32)
    zeros32 = jnp.zeros((N_ACC, 2 * H2), jnp.float32)
    zeros16 = jnp.zeros((N_ACC, 16), jnp.float32)
    ones16 = jnp.zeros((128, 16), jnp.float32).at[:, 0].set(1.0)
    wn = jnp.concatenate([Wm_nbr, Wv_nbr], axis=1)

    sc_deg, sc_agg64, sc_agg32 = _sc_kernels()
    h = _tc_mm(x, W_gc1)
    degp = sc_deg(dst_p, ones16, zeros16)
    g = _tc_scale(degp, h)
    agg1p = sc_agg64(src_p, dst_p, g, zeros64)
    h1, t2 = _tc_h1(degp, agg1p, g, b_gc1.reshape(1, H1), wn)
    agg2p = sc_agg32(src_p, dst_p, t2, zeros32)
    mu, logvar = _tc_head(h1, agg2p, Wm_root, bm.reshape(1, H2),
                          Wv_root, bv.reshape(1, H2))
    adj = _tc_dec(mu)
    return (adj, mu, mu, logvar)


# speedup vs baseline: 1.0466x; 1.0368x over previous
import functools

import jax
import jax.numpy as jnp
from jax import lax
from jax.experimental import pallas as pl
from jax.experimental.pallas import tpu as pltpu
from jax.experimental.pallas import tpu_sc as plsc

N = 10000
E = 320000
D_IN = 128
H1 = 64
H2 = 16

NC = 2
NS = 16
NW = NC * NS
E_PAD = 327680
EROWS = E_PAD // 128
EROWS_W = EROWS // NW
CHUNKS = EROWS_W // 8
N_ACC = 10112
ROWS_T = N_ACC // NS
SPLIT_C0 = 80
SPLIT_C1 = 80


def _sc_deg_body(dst_hbm, ones_hbm, zeros_hbm, out_hbm, idx_v, ones_v, acc, sem):
    c = lax.axis_index("c")
    s = lax.axis_index("s")
    wid = s * NC + c
    pltpu.sync_copy(zeros_hbm.at[pl.ds(s * ROWS_T, ROWS_T)],
                    acc.at[pl.ds(s * ROWS_T, ROWS_T)])
    pltpu.sync_copy(ones_hbm, ones_v)
    plsc.subcore_barrier()
    pltpu.sync_copy(dst_hbm.at[pl.ds(wid * EROWS_W, EROWS_W)], idx_v)

    def body(j, carry):
        pltpu.sync_copy(ones_v, acc.at[idx_v.at[j]], add=True)
        return carry

    lax.fori_loop(0, EROWS_W, body, 0)
    plsc.subcore_barrier()
    pltpu.sync_copy(acc.at[pl.ds(s * ROWS_T, ROWS_T)],
                    out_hbm.at[c, pl.ds(s * ROWS_T, ROWS_T)])


def _sc_agg_body(src_hbm, dst_hbm, table_hbm, zeros_hbm, out_hbm,
                 isrc, idst, rows_v, acc,
                 g0, g1, g2, g3, s0, s1, s2, s3):
    c = lax.axis_index("c")
    s = lax.axis_index("s")
    gsem = (g0, g1, g2, g3)
    ssem = (s0, s1, s2, s3)
    pltpu.sync_copy(zeros_hbm.at[pl.ds(s * ROWS_T, ROWS_T)],
                    acc.at[pl.ds(s * ROWS_T, ROWS_T)])
    plsc.subcore_barrier()

    def fire_gather(k, b):
        pltpu.async_copy(table_hbm.at[isrc.at[k]],
                         rows_v.at[pl.ds(b * 128, 128)], gsem[b])

    def gather_done(k, b):
        pltpu.make_async_copy(table_hbm.at[isrc.at[k]],
                              rows_v.at[pl.ds(b * 128, 128)], gsem[b]).wait()

    def fire_scatter(k, b):
        pltpu.async_copy(rows_v.at[pl.ds(b * 128, 128)],
                         acc.at[idst.at[k]], ssem[b], add=True)

    def scatter_done(k, b):
        pltpu.make_async_copy(rows_v.at[pl.ds(b * 128, 128)],
                              acc.at[idst.at[k]], ssem[b]).wait()

    def pipeline(hbm_base, n_rows):
        pltpu.sync_copy(src_hbm.at[pl.ds(hbm_base, n_rows)],
                        isrc.at[pl.ds(0, n_rows)])
        pltpu.sync_copy(dst_hbm.at[pl.ds(hbm_base, n_rows)],
                        idst.at[pl.ds(0, n_rows)])
        for b in range(2):
            fire_gather(b, b)

        def round_(r, carry):
            for b in range(4):
                k = 4 * r + b
                b2 = (b + 2) % 4
                gather_done(k, b)
                fire_scatter(k, b)

                @pl.when(k >= 2)
                def _():
                    scatter_done(k - 2, b2)

                @pl.when(k + 2 < n_rows)
                def _():
                    fire_gather(k + 2, b2)

            return carry

        lax.fori_loop(0, n_rows // 4, round_, 0)
        for k in range(n_rows - 2, n_rows):
            scatter_done(k, k % 4)

    if SPLIT_C0 > 0:
        @pl.when(c == 0)
        def _():
            pipeline(s * 160, SPLIT_C0)
    if SPLIT_C1 > 0:
        @pl.when(c == 1)
        def _():
            pipeline(s * 160 + SPLIT_C0, SPLIT_C1)
    plsc.subcore_barrier()
    pltpu.sync_copy(acc.at[pl.ds(s * ROWS_T, ROWS_T)],
                    out_hbm.at[c, pl.ds(s * ROWS_T, ROWS_T)])


@functools.lru_cache(maxsize=None)
def _sc_kernels():
    mesh = plsc.VectorSubcoreMesh(core_axis_name="c", subcore_axis_name="s")
    params = pltpu.CompilerParams(use_tc_tiling_on_sc=False)
    sc_deg = pl.kernel(
        _sc_deg_body,
        out_type=jax.ShapeDtypeStruct((NC, N_ACC, 16), jnp.float32),
        mesh=mesh,
        compiler_params=params,
        scratch_types=[
            pltpu.VMEM((EROWS_W, 128), jnp.int32),
            pltpu.VMEM((128, 16), jnp.float32),
            pltpu.VMEM_SHARED((N_ACC, 16), jnp.float32),
            pltpu.SemaphoreType.DMA,
        ],
    )
    def make_agg(width):
        return pl.kernel(
            _sc_agg_body,
            out_type=jax.ShapeDtypeStruct((NC, N_ACC, width), jnp.float32),
            mesh=mesh,
            compiler_params=params,
            scratch_types=[
                pltpu.VMEM((80, 128), jnp.int32),
                pltpu.VMEM((80, 128), jnp.int32),
                pltpu.VMEM((512, width), jnp.float32),
                pltpu.VMEM_SHARED((N_ACC, width), jnp.float32),
            ] + [pltpu.SemaphoreType.DMA] * 8,
        )

    return sc_deg, make_agg(H1), make_agg(2 * H2)


def _dinv_from(degp, n_rows):
    deg = degp[0][:n_rows, 0:1] + degp[1][:n_rows, 0:1] + 1.0
    return lax.rsqrt(deg)


def _tc_pre_body(degp_ref, x_ref, w_ref, g_ref):
    dinv = _dinv_from(degp_ref[...], N)
    h = jnp.dot(x_ref[...], w_ref[...],
                preferred_element_type=jnp.float32)
    g_ref[...] = h * dinv


def _tc_h1_body(degp_ref, aggp_ref, g_ref, b_ref, wn_ref, h1_ref, t2_ref):
    dinv = _dinv_from(degp_ref[...], N)
    aggp = aggp_ref[...]
    agg = aggp[0][:N] + aggp[1][:N]
    h1 = dinv * (agg + g_ref[...]) + b_ref[...]
    h1_ref[...] = h1
    t2_ref[...] = jnp.dot(h1, wn_ref[...], preferred_element_type=jnp.float32)


def _tc_head_body(h1_ref, aggp_ref, wmr_ref, bm_ref,
                  wvr_ref, bv_ref, mu_ref, lv_ref):
    aggp = aggp_ref[...]
    agg = aggp[0][:N] + aggp[1][:N]
    h1 = h1_ref[...]
    mu_ref[...] = (jnp.dot(h1, wmr_ref[...], preferred_element_type=jnp.float32)
                   + agg[:, :H2] + bm_ref[...])
    lv_ref[...] = (jnp.dot(h1, wvr_ref[...], preferred_element_type=jnp.float32)
                   + agg[:, H2:] + bv_ref[...])


_DEC_B = 1024


def _tc_dec_body(a_ref, b_ref, o_ref):
    v = lax.dot_general(a_ref[...], b_ref[...],
                        (((1,), (1,)), ((), ())),
                        preferred_element_type=jnp.float32)
    o_ref[...] = jax.nn.sigmoid(v)


def _tc_pre(degp, x, w):
    return pl.pallas_call(
        _tc_pre_body,
        out_shape=jax.ShapeDtypeStruct((N, H1), jnp.float32),
    )(degp, x, w)


def _tc_h1(degp, aggp, g, b, wn):
    return pl.pallas_call(
        _tc_h1_body,
        out_shape=(jax.ShapeDtypeStruct((N, H1), jnp.float32),
                   jax.ShapeDtypeStruct((N, 2 * H2), jnp.float32)),
    )(degp, aggp, g, b, wn)


def _tc_head(h1, aggp, wmr, bm, wvr, bv):
    return pl.pallas_call(
        _tc_head_body,
        out_shape=(jax.ShapeDtypeStruct((N, H2), jnp.float32),
                   jax.ShapeDtypeStruct((N, H2), jnp.float32)),
    )(h1, aggp, wmr, bm, wvr, bv)


def _tc_dec(z):
    grid = (pl.cdiv(N, _DEC_B), pl.cdiv(N, _DEC_B))
    return pl.pallas_call(
        _tc_dec_body,
        grid=grid,
        in_specs=[
            pl.BlockSpec((_DEC_B, H2), lambda i, j: (i, 0)),
            pl.BlockSpec((_DEC_B, H2), lambda i, j: (j, 0)),
        ],
        out_specs=pl.BlockSpec((_DEC_B, _DEC_B), lambda i, j: (i, j)),
        out_shape=jax.ShapeDtypeStruct((N, N), jnp.float32),
    )(z, z)


def kernel(x, edge_index, W_gc1, b_gc1, Wm_root, Wm_nbr, bm, Wv_root, Wv_nbr, bv):
    src = edge_index[0]
    dst = edge_index[1]
    pad = E_PAD - E
    src_p = jnp.concatenate(
        [src, jnp.zeros((pad,), jnp.int32)]).reshape(EROWS, 128)
    dst_p = jnp.concatenate(
        [dst, jnp.full((pad,), N, jnp.int32)]).reshape(EROWS, 128)
    zeros64 = jnp.zeros((N_ACC, H1), jnp.float32)
    zeros32 = jnp.zeros((N_ACC, 2 * H2), jnp.float32)
    zeros16 = jnp.zeros((N_ACC, 16), jnp.float32)
    ones16 = jnp.zeros((128, 16), jnp.float32).at[:, 0].set(1.0)
    wn = jnp.concatenate([Wm_nbr, Wv_nbr], axis=1)

    sc_deg, sc_agg64, sc_agg32 = _sc_kernels()
    degp = sc_deg(dst_p, ones16, zeros16)
    g = _tc_pre(degp, x, W_gc1)
    agg1p = sc_agg64(src_p, dst_p, g, zeros64)
    h1, t2 = _tc_h1(degp, agg1p, g, b_gc1.reshape(1, H1), wn)
    agg2p = sc_agg32(src_p, dst_p, t2, zeros32)
    mu, logvar = _tc_head(h1, agg2p, Wm_root, bm.reshape(1, H2),
                          Wv_root, bv.reshape(1, H2))
    adj = _tc_dec(mu)
    return (adj, mu, mu, logvar)


# speedup vs baseline: 1.1016x; 1.0526x over previous
import functools

import jax
import jax.numpy as jnp
from jax import lax
from jax.experimental import pallas as pl
from jax.experimental.pallas import tpu as pltpu
from jax.experimental.pallas import tpu_sc as plsc

N = 10000
E = 320000
D_IN = 128
H1 = 64
H2 = 16

NC = 2
NS = 16
NW = NC * NS
E_PAD = 327680
EROWS = E_PAD // 128
EROWS_W = EROWS // NW
CHUNKS = EROWS_W // 8
N_ACC = 10112
ROWS_T = N_ACC // NS
SPLIT_C0 = 80
SPLIT_C1 = 80


def _sc_deg_body(dst_hbm, ones_hbm, zeros_hbm, out_hbm, idx_v, ones_v, acc, sem):
    c = lax.axis_index("c")
    s = lax.axis_index("s")
    wid = s * NC + c
    pltpu.sync_copy(zeros_hbm.at[pl.ds(s * ROWS_T, ROWS_T)],
                    acc.at[pl.ds(s * ROWS_T, ROWS_T)])
    pltpu.sync_copy(ones_hbm, ones_v)
    plsc.subcore_barrier()
    pltpu.sync_copy(dst_hbm.at[pl.ds(wid * EROWS_W, EROWS_W)], idx_v)

    def body(j, carry):
        pltpu.sync_copy(ones_v, acc.at[idx_v.at[j]], add=True)
        return carry

    lax.fori_loop(0, EROWS_W, body, 0)
    plsc.subcore_barrier()
    pltpu.sync_copy(acc.at[pl.ds(s * ROWS_T, ROWS_T)],
                    out_hbm.at[c, pl.ds(s * ROWS_T, ROWS_T)])


def _sc_agg_body(src_hbm, dst_hbm, table_hbm, zeros_hbm, out_hbm,
                 isrc, idst, rows_v, acc,
                 g0, g1, g2, g3, s0, s1, s2, s3):
    c = lax.axis_index("c")
    s = lax.axis_index("s")
    gsem = (g0, g1, g2, g3)
    ssem = (s0, s1, s2, s3)
    pltpu.sync_copy(zeros_hbm.at[pl.ds(s * ROWS_T, ROWS_T)],
                    acc.at[pl.ds(s * ROWS_T, ROWS_T)])
    plsc.subcore_barrier()

    def fire_gather(k, b):
        pltpu.async_copy(table_hbm.at[isrc.at[k]],
                         rows_v.at[pl.ds(b * 128, 128)], gsem[b])

    def gather_done(k, b):
        pltpu.make_async_copy(table_hbm.at[isrc.at[k]],
                              rows_v.at[pl.ds(b * 128, 128)], gsem[b]).wait()

    def fire_scatter(k, b):
        pltpu.async_copy(rows_v.at[pl.ds(b * 128, 128)],
                         acc.at[idst.at[k]], ssem[b], add=True)

    def scatter_done(k, b):
        pltpu.make_async_copy(rows_v.at[pl.ds(b * 128, 128)],
                              acc.at[idst.at[k]], ssem[b]).wait()

    def pipeline(hbm_base, n_rows):
        pltpu.sync_copy(src_hbm.at[pl.ds(hbm_base, n_rows)],
                        isrc.at[pl.ds(0, n_rows)])
        pltpu.sync_copy(dst_hbm.at[pl.ds(hbm_base, n_rows)],
                        idst.at[pl.ds(0, n_rows)])
        for b in range(2):
            fire_gather(b, b)

        def round_(r, carry):
            for b in range(4):
                k = 4 * r + b
                b2 = (b + 2) % 4
                gather_done(k, b)
                fire_scatter(k, b)

                @pl.when(k >= 2)
                def _():
                    scatter_done(k - 2, b2)

                @pl.when(k + 2 < n_rows)
                def _():
                    fire_gather(k + 2, b2)

            return carry

        lax.fori_loop(0, n_rows // 4, round_, 0)
        for k in range(n_rows - 2, n_rows):
            scatter_done(k, k % 4)

    if SPLIT_C0 > 0:
        @pl.when(c == 0)
        def _():
            pipeline(s * 160, SPLIT_C0)
    if SPLIT_C1 > 0:
        @pl.when(c == 1)
        def _():
            pipeline(s * 160 + SPLIT_C0, SPLIT_C1)
    plsc.subcore_barrier()
    pltpu.sync_copy(acc.at[pl.ds(s * ROWS_T, ROWS_T)],
                    out_hbm.at[c, pl.ds(s * ROWS_T, ROWS_T)])


@functools.lru_cache(maxsize=None)
def _sc_kernels():
    mesh = plsc.VectorSubcoreMesh(core_axis_name="c", subcore_axis_name="s")
    params = pltpu.CompilerParams(use_tc_tiling_on_sc=False)
    sc_deg = pl.kernel(
        _sc_deg_body,
        out_type=jax.ShapeDtypeStruct((NC, N_ACC, 16), jnp.float32),
        mesh=mesh,
        compiler_params=params,
        scratch_types=[
            pltpu.VMEM((EROWS_W, 128), jnp.int32),
            pltpu.VMEM((128, 16), jnp.float32),
            pltpu.VMEM_SHARED((N_ACC, 16), jnp.float32),
            pltpu.SemaphoreType.DMA,
        ],
    )
    def make_agg(width):
        return pl.kernel(
            _sc_agg_body,
            out_type=jax.ShapeDtypeStruct((NC, N_ACC, width), jnp.float32),
            mesh=mesh,
            compiler_params=params,
            scratch_types=[
                pltpu.VMEM((80, 128), jnp.int32),
                pltpu.VMEM((80, 128), jnp.int32),
                pltpu.VMEM((512, width), jnp.float32),
                pltpu.VMEM_SHARED((N_ACC, width), jnp.float32),
            ] + [pltpu.SemaphoreType.DMA] * 8,
        )

    return sc_deg, make_agg(H1), make_agg(2 * H2)


def _dinv_from(degp, n_rows):
    deg = degp[0][:n_rows, 0:1] + degp[1][:n_rows, 0:1] + 1.0
    return lax.rsqrt(deg)


def _tc_pre_body(degp_ref, x_ref, w_ref, g_ref):
    dinv = _dinv_from(degp_ref[...], N)
    h = jnp.dot(x_ref[...], w_ref[...],
                preferred_element_type=jnp.float32)
    g_ref[...] = h * dinv


def _tc_h1_body(degp_ref, aggp_ref, g_ref, b_ref, wn_ref, h1_ref, t2_ref):
    dinv = _dinv_from(degp_ref[...], N)
    aggp = aggp_ref[...]
    agg = aggp[0][:N] + aggp[1][:N]
    h1 = dinv * (agg + g_ref[...]) + b_ref[...]
    h1_ref[...] = h1
    t2_ref[...] = jnp.dot(h1, wn_ref[...], preferred_element_type=jnp.float32)


def _tc_head_body(h1_ref, aggp_ref, wmr_ref, bm_ref,
                  wvr_ref, bv_ref, mu_ref, lv_ref):
    aggp = aggp_ref[...]
    agg = aggp[0][:N] + aggp[1][:N]
    h1 = h1_ref[...]
    mu_ref[...] = (jnp.dot(h1, wmr_ref[...], preferred_element_type=jnp.float32)
                   + agg[:, :H2] + bm_ref[...])
    lv_ref[...] = (jnp.dot(h1, wvr_ref[...], preferred_element_type=jnp.float32)
                   + agg[:, H2:] + bv_ref[...])


_DEC_B = 256


def _tc_dec_body(a_ref, b_ref, o_ref):
    v = lax.dot_general(a_ref[...], b_ref[...],
                        (((1,), (1,)), ((), ())),
                        preferred_element_type=jnp.float32)
    o_ref[...] = jax.nn.sigmoid(v)


def _tc_pre(degp, x, w):
    return pl.pallas_call(
        _tc_pre_body,
        out_shape=jax.ShapeDtypeStruct((N, H1), jnp.float32),
    )(degp, x, w)


def _tc_h1(degp, aggp, g, b, wn):
    return pl.pallas_call(
        _tc_h1_body,
        out_shape=(jax.ShapeDtypeStruct((N, H1), jnp.float32),
                   jax.ShapeDtypeStruct((N, 2 * H2), jnp.float32)),
    )(degp, aggp, g, b, wn)


def _tc_head(h1, aggp, wmr, bm, wvr, bv):
    return pl.pallas_call(
        _tc_head_body,
        out_shape=(jax.ShapeDtypeStruct((N, H2), jnp.float32),
                   jax.ShapeDtypeStruct((N, H2), jnp.float32)),
    )(h1, aggp, wmr, bm, wvr, bv)


def _tc_dec(z):
    grid = (pl.cdiv(N, _DEC_B),)
    return pl.pallas_call(
        _tc_dec_body,
        grid=grid,
        in_specs=[
            pl.BlockSpec((_DEC_B, H2), lambda i: (i, 0)),
            pl.BlockSpec((N, H2), lambda i: (0, 0)),
        ],
        out_specs=pl.BlockSpec((_DEC_B, N), lambda i: (i, 0)),
        out_shape=jax.ShapeDtypeStruct((N, N), jnp.float32),
    )(z, z)


def kernel(x, edge_index, W_gc1, b_gc1, Wm_root, Wm_nbr, bm, Wv_root, Wv_nbr, bv):
    src = edge_index[0]
    dst = edge_index[1]
    pad = E_PAD - E
    src_p = jnp.concatenate(
        [src, jnp.zeros((pad,), jnp.int32)]).reshape(EROWS, 128)
    dst_p = jnp.concatenate(
        [dst, jnp.full((pad,), N, jnp.int32)]).reshape(EROWS, 128)
    zeros64 = jnp.zeros((N_ACC, H1), jnp.float32)
    zeros32 = jnp.zeros((N_ACC, 2 * H2), jnp.float32)
    zeros16 = jnp.zeros((N_ACC, 16), jnp.float32)
    ones16 = jnp.zeros((128, 16), jnp.float32).at[:, 0].set(1.0)
    wn = jnp.concatenate([Wm_nbr, Wv_nbr], axis=1)

    sc_deg, sc_agg64, sc_agg32 = _sc_kernels()
    degp = sc_deg(dst_p, ones16, zeros16)
    g = _tc_pre(degp, x, W_gc1)
    agg1p = sc_agg64(src_p, dst_p, g, zeros64)
    h1, t2 = _tc_h1(degp, agg1p, g, b_gc1.reshape(1, H1), wn)
    agg2p = sc_agg32(src_p, dst_p, t2, zeros32)
    mu, logvar = _tc_head(h1, agg2p, Wm_root, bm.reshape(1, H2),
                          Wv_root, bv.reshape(1, H2))
    adj = _tc_dec(mu)
    return (adj, mu, mu, logvar)
